# baseline (speedup 1.0000x reference)
import functools

import jax
import jax.numpy as jnp
from jax.experimental import pallas as pl
from jax.experimental.pallas import tpu as pltpu


_S = 8
_NK = 8
_NBUF = 4


def _rnn_kernel(x_hbm, h0_ref, wih_ref, bh_ref, wio_ref, bo_ref,
                out_ref, hfin_ref,
                xbuf, q_scr, a_scr, wxf_scr, whf_scr, b_scr, insem,
                *, isz, hsz, osz, bb):
    f32 = jnp.float32
    bf16 = jnp.bfloat16
    g = pl.program_id(0)
    dot = lambda u, v: jnp.dot(u, v, preferred_element_type=f32)
    db = lambda u, v: dot(u, v).astype(bf16)

    def dma_in(slot, k):
        pltpu.make_async_copy(
            x_hbm.at[pl.ds(k * _S, _S), pl.ds(g * bb, bb), :],
            xbuf.at[slot], insem.at[slot]).start()

    def wait_in(slot):
        pltpu.make_async_copy(
            x_hbm.at[pl.ds(0, _S), pl.ds(0, bb), :],
            xbuf.at[slot], insem.at[slot]).wait()

    for b in range(_NBUF):
        dma_in(b, b)

    wxh = wih_ref[0:isz, :]
    A = wih_ref[isz:, :]
    Ab = A.astype(bf16)
    A2 = db(Ab, Ab)
    A3 = db(A2, Ab)
    A4 = db(A2, A2)
    A5 = db(A4, Ab)
    A6 = db(A4, A2)
    A7 = db(A4, A3)
    A8 = db(A4, A4)
    pows = [None, Ab, A2, A3, A4, A5, A6, A7, A8]

    wxhb = wxh.astype(bf16)
    q_scr[0:isz, :] = wxhb
    for j in range(1, _S):
        q_scr[j * isz:(j + 1) * isz, :] = db(wxhb, pows[j])

    a_scr[0:hsz, :] = A8
    a_scr[hsz:, :] = A7

    bh = bh_ref[...]
    b7 = bh
    v = bh
    for j in range(1, _S):
        v = dot(v, A)
        if j < _S - 1:
            b7 = b7 + v
    b_scr[0:1, 0:hsz] = b7 + v
    b_scr[1:2, 0:hsz] = b7
    b_scr[2:3, 0:hsz] = bh
    b_scr[2:3, hsz:] = bo_ref[...]

    wxf_scr[:, 0:hsz] = wxhb
    wxf_scr[:, hsz:] = wio_ref[0:isz, :].astype(bf16)
    whf_scr[:, 0:hsz] = Ab
    whf_scr[:, hsz:] = wio_ref[isz:, :].astype(bf16)

    def xdot(slot, t, j):
        x = xbuf[slot, t].astype(bf16)
        q = q_scr[j * isz:(j + 1) * isz, :]
        return jnp.dot(x, q, preferred_element_type=f32)

    h = h0_ref[...]
    for k in range(_NK - 1):
        slot = k % _NBUF
        wait_in(slot)
        s = jnp.dot(h.astype(bf16), a_scr[0:hsz, :], preferred_element_type=f32)
        for j in range(_S):
            s = s + xdot(slot, j, _S - 1 - j)
        h = s + b_scr[0:1, 0:hsz]
        if k + _NBUF < _NK:
            dma_in(slot, k + _NBUF)

    slot = (_NK - 1) % _NBUF
    wait_in(slot)
    s = jnp.dot(h.astype(bf16), a_scr[hsz:, :], preferred_element_type=f32)
    for j in range(_S - 1):
        s = s + xdot(slot, j, _S - 2 - j)
    h = s + b_scr[1:2, 0:hsz]

    xl = xbuf[slot, _S - 1].astype(bf16)
    acc = (jnp.dot(xl, wxf_scr[...], preferred_element_type=f32)
           + jnp.dot(h.astype(bf16), whf_scr[...], preferred_element_type=f32)
           + b_scr[2:3, :])
    hfin_ref[...] = acc[:, 0:hsz]
    logits = acc[:, hsz:]
    m = jnp.max(logits, axis=1, keepdims=True)
    sh = logits - m
    out_ref[...] = sh - jnp.log(jnp.sum(jnp.exp(sh), axis=1, keepdims=True))


@jax.jit
def _rnn_fused(x_seq, h0, w_i2h_t, b_i2h, w_i2o_t, b_i2o):
    T, B, I = x_seq.shape
    H = h0.shape[1]
    O = w_i2o_t.shape[1]
    f32, bf16 = jnp.float32, jnp.bfloat16
    Nf = H + O

    nb = 2 if (B % 16 == 0) else 1
    bb = B // nb

    out, hfin = pl.pallas_call(
        functools.partial(_rnn_kernel, isz=I, hsz=H, osz=O, bb=bb),
        grid=(nb,),
        in_specs=[
            pl.BlockSpec(memory_space=pl.ANY),
            pl.BlockSpec((bb, H), lambda g: (g, 0)),
            pl.BlockSpec((I + H, H), lambda g: (0, 0)),
            pl.BlockSpec((1, H), lambda g: (0, 0)),
            pl.BlockSpec((I + H, O), lambda g: (0, 0)),
            pl.BlockSpec((1, O), lambda g: (0, 0)),
        ],
        out_specs=(
            pl.BlockSpec((bb, O), lambda g: (g, 0)),
            pl.BlockSpec((bb, H), lambda g: (g, 0)),
        ),
        out_shape=(
            jax.ShapeDtypeStruct((B, O), f32),
            jax.ShapeDtypeStruct((B, H), f32),
        ),
        scratch_shapes=[
            pltpu.VMEM((_NBUF, _S, bb, I), f32),
            pltpu.VMEM((_S * I, H), bf16),
            pltpu.VMEM((2 * H, H), bf16),
            pltpu.VMEM((I, Nf), bf16),
            pltpu.VMEM((H, Nf), bf16),
            pltpu.VMEM((8, Nf), f32),
            pltpu.SemaphoreType.DMA((_NBUF,)),
        ],
        compiler_params=pltpu.CompilerParams(
            dimension_semantics=("parallel",),
        ),
    )(x_seq.astype(f32), h0.astype(f32),
      w_i2h_t.astype(f32), b_i2h.astype(f32),
      w_i2o_t.astype(f32), b_i2o.astype(f32))

    return out, hfin


def kernel(x_seq, h0, w_i2h_t, b_i2h, w_i2o_t, b_i2o):
    return _rnn_fused(x_seq, h0, w_i2h_t, b_i2h, w_i2o_t, b_i2o)


# speedup vs baseline: 1.0072x; 1.0072x over previous
import functools

import jax
import jax.numpy as jnp
from jax.experimental import pallas as pl
from jax.experimental.pallas import tpu as pltpu


_S = 8
_NK = 8
_NBUF = 8


def _rnn_kernel(x_hbm, h0_ref, wih_ref, bh_ref, wio_ref, bo_ref,
                out_ref, hfin_ref,
                xbuf, q_scr, a_scr, wxf_scr, whf_scr, b_scr, insem,
                *, isz, hsz, osz, bb):
    f32 = jnp.float32
    bf16 = jnp.bfloat16
    g = pl.program_id(0)
    dot = lambda u, v: jnp.dot(u, v, preferred_element_type=f32)
    db = lambda u, v: dot(u, v).astype(bf16)

    def dma_in(slot, k):
        pltpu.make_async_copy(
            x_hbm.at[pl.ds(k * _S, _S), pl.ds(g * bb, bb), :],
            xbuf.at[slot], insem.at[slot]).start()

    def wait_in(slot):
        pltpu.make_async_copy(
            x_hbm.at[pl.ds(0, _S), pl.ds(0, bb), :],
            xbuf.at[slot], insem.at[slot]).wait()

    for b in range(_NBUF):
        dma_in(b, b)

    wxh = wih_ref[0:isz, :]
    A = wih_ref[isz:, :]
    Ab = A.astype(bf16)
    A2 = db(Ab, Ab)
    A3 = db(A2, Ab)
    A4 = db(A2, A2)
    A5 = db(A4, Ab)
    A6 = db(A4, A2)
    A7 = db(A4, A3)
    A8 = db(A4, A4)
    pows = [None, Ab, A2, A3, A4, A5, A6, A7, A8]

    wxhb = wxh.astype(bf16)
    q_scr[0:isz, :] = wxhb
    for j in range(1, _S):
        q_scr[j * isz:(j + 1) * isz, :] = db(wxhb, pows[j])

    a_scr[0:hsz, :] = A8
    a_scr[hsz:, :] = A7

    bh = bh_ref[...]
    b7 = bh
    v = bh
    for j in range(1, _S):
        v = dot(v, A)
        if j < _S - 1:
            b7 = b7 + v
    b_scr[0:1, 0:hsz] = b7 + v
    b_scr[1:2, 0:hsz] = b7
    b_scr[2:3, 0:hsz] = bh
    b_scr[2:3, hsz:] = bo_ref[...]

    wxf_scr[:, 0:hsz] = wxhb
    wxf_scr[:, hsz:] = wio_ref[0:isz, :].astype(bf16)
    whf_scr[:, 0:hsz] = Ab
    whf_scr[:, hsz:] = wio_ref[isz:, :].astype(bf16)

    def xdot(slot, t, j):
        x = xbuf[slot, t].astype(bf16)
        q = q_scr[j * isz:(j + 1) * isz, :]
        return jnp.dot(x, q, preferred_element_type=f32)

    h = h0_ref[...]
    for k in range(_NK - 1):
        slot = k % _NBUF
        wait_in(slot)
        s = jnp.dot(h.astype(bf16), a_scr[0:hsz, :], preferred_element_type=f32)
        for j in range(_S):
            s = s + xdot(slot, j, _S - 1 - j)
        h = s + b_scr[0:1, 0:hsz]
        if k + _NBUF < _NK:
            dma_in(slot, k + _NBUF)

    slot = (_NK - 1) % _NBUF
    wait_in(slot)
    s = jnp.dot(h.astype(bf16), a_scr[hsz:, :], preferred_element_type=f32)
    for j in range(_S - 1):
        s = s + xdot(slot, j, _S - 2 - j)
    h = s + b_scr[1:2, 0:hsz]

    xl = xbuf[slot, _S - 1].astype(bf16)
    acc = (jnp.dot(xl, wxf_scr[...], preferred_element_type=f32)
           + jnp.dot(h.astype(bf16), whf_scr[...], preferred_element_type=f32)
           + b_scr[2:3, :])
    hfin_ref[...] = acc[:, 0:hsz]
    logits = acc[:, hsz:]
    m = jnp.max(logits, axis=1, keepdims=True)
    sh = logits - m
    out_ref[...] = sh - jnp.log(jnp.sum(jnp.exp(sh), axis=1, keepdims=True))


@jax.jit
def _rnn_fused(x_seq, h0, w_i2h_t, b_i2h, w_i2o_t, b_i2o):
    T, B, I = x_seq.shape
    H = h0.shape[1]
    O = w_i2o_t.shape[1]
    f32, bf16 = jnp.float32, jnp.bfloat16
    Nf = H + O

    nb = 2 if (B % 16 == 0) else 1
    bb = B // nb

    out, hfin = pl.pallas_call(
        functools.partial(_rnn_kernel, isz=I, hsz=H, osz=O, bb=bb),
        grid=(nb,),
        in_specs=[
            pl.BlockSpec(memory_space=pl.ANY),
            pl.BlockSpec((bb, H), lambda g: (g, 0)),
            pl.BlockSpec((I + H, H), lambda g: (0, 0)),
            pl.BlockSpec((1, H), lambda g: (0, 0)),
            pl.BlockSpec((I + H, O), lambda g: (0, 0)),
            pl.BlockSpec((1, O), lambda g: (0, 0)),
        ],
        out_specs=(
            pl.BlockSpec((bb, O), lambda g: (g, 0)),
            pl.BlockSpec((bb, H), lambda g: (g, 0)),
        ),
        out_shape=(
            jax.ShapeDtypeStruct((B, O), f32),
            jax.ShapeDtypeStruct((B, H), f32),
        ),
        scratch_shapes=[
            pltpu.VMEM((_NBUF, _S, bb, I), f32),
            pltpu.VMEM((_S * I, H), bf16),
            pltpu.VMEM((2 * H, H), bf16),
            pltpu.VMEM((I, Nf), bf16),
            pltpu.VMEM((H, Nf), bf16),
            pltpu.VMEM((8, Nf), f32),
            pltpu.SemaphoreType.DMA((_NBUF,)),
        ],
        compiler_params=pltpu.CompilerParams(
            dimension_semantics=("parallel",),
        ),
    )(x_seq.astype(f32), h0.astype(f32),
      w_i2h_t.astype(f32), b_i2h.astype(f32),
      w_i2o_t.astype(f32), b_i2o.astype(f32))

    return out, hfin


def kernel(x_seq, h0, w_i2h_t, b_i2h, w_i2o_t, b_i2o):
    return _rnn_fused(x_seq, h0, w_i2h_t, b_i2h, w_i2o_t, b_i2o)


# speedup vs baseline: 1.0074x; 1.0002x over previous
import functools

import jax
import jax.numpy as jnp
from jax.experimental import pallas as pl
from jax.experimental.pallas import tpu as pltpu


_S = 8
_NK = 8
_NBUF = 8


def _rnn_kernel(x_hbm, h0_ref, wih_ref, bh_ref, wio_ref, bo_ref,
                out_ref, hfin_ref,
                xbuf, q_scr, a_scr, wxf_scr, whf_scr, b_scr, insem,
                *, isz, hsz, osz, bb):
    f32 = jnp.float32
    bf16 = jnp.bfloat16
    g = pl.program_id(0)
    dot = lambda u, v: jnp.dot(u, v, preferred_element_type=f32)
    db = lambda u, v: dot(u, v).astype(bf16)

    def dma_in(slot, k):
        pltpu.make_async_copy(
            x_hbm.at[pl.ds(k * _S, _S), pl.ds(g * bb, bb), :],
            xbuf.at[slot], insem.at[slot]).start()

    def wait_in(slot):
        pltpu.make_async_copy(
            x_hbm.at[pl.ds(0, _S), pl.ds(0, bb), :],
            xbuf.at[slot], insem.at[slot]).wait()

    for b in range(_NBUF):
        dma_in(b, b)

    wxh = wih_ref[0:isz, :]
    A = wih_ref[isz:, :]
    Ab = A.astype(bf16)
    A2 = db(Ab, Ab)
    A3 = db(A2, Ab)
    A4 = db(A2, A2)
    A5 = db(A4, Ab)
    A6 = db(A4, A2)
    A7 = db(A4, A3)
    A8 = db(A4, A4)
    pows = [None, Ab, A2, A3, A4, A5, A6, A7, A8]

    wxhb = wxh.astype(bf16)
    q_scr[0:isz, :] = wxhb
    for j in range(1, _S):
        q_scr[j * isz:(j + 1) * isz, :] = db(wxhb, pows[j])

    a_scr[0:hsz, :] = A8
    a_scr[hsz:, :] = A7

    bh = bh_ref[...]
    b7 = bh
    v = bh
    for j in range(1, _S):
        v = dot(v, A)
        if j < _S - 1:
            b7 = b7 + v
    b_scr[0:1, 0:hsz] = b7 + v
    b_scr[1:2, 0:hsz] = b7
    b_scr[2:3, 0:hsz] = bh
    b_scr[2:3, hsz:] = bo_ref[...]

    wxf_scr[:, 0:hsz] = wxhb
    wxf_scr[:, hsz:] = wio_ref[0:isz, :].astype(bf16)
    whf_scr[:, 0:hsz] = Ab
    whf_scr[:, hsz:] = wio_ref[isz:, :].astype(bf16)

    def xdot(slot, t, j):
        x = xbuf[slot, t].astype(bf16)
        q = q_scr[j * isz:(j + 1) * isz, :]
        return jnp.dot(x, q, preferred_element_type=f32)

    h = h0_ref[...]
    for k in range(_NK - 1):
        slot = k % _NBUF
        wait_in(slot)
        s = jnp.dot(h.astype(bf16), a_scr[0:hsz, :], preferred_element_type=f32)
        for j in range(_S):
            s = s + xdot(slot, j, _S - 1 - j)
        h = s + b_scr[0:1, 0:hsz]
        if k + _NBUF < _NK:
            dma_in(slot, k + _NBUF)

    slot = (_NK - 1) % _NBUF
    wait_in(slot)
    s = jnp.dot(h.astype(bf16), a_scr[hsz:, :], preferred_element_type=f32)
    for j in range(_S - 1):
        s = s + xdot(slot, j, _S - 2 - j)
    h = s + b_scr[1:2, 0:hsz]

    xl = xbuf[slot, _S - 1].astype(bf16)
    acc = (jnp.dot(xl, wxf_scr[...], preferred_element_type=f32)
           + jnp.dot(h.astype(bf16), whf_scr[...], preferred_element_type=f32)
           + b_scr[2:3, :])
    hfin_ref[...] = acc[:, 0:hsz]
    logits = acc[:, hsz:]
    m = jnp.max(logits, axis=1, keepdims=True)
    sh = logits - m
    out_ref[...] = sh - jnp.log(jnp.sum(jnp.exp(sh), axis=1, keepdims=True))


@jax.jit
def _rnn_fused(x_seq, h0, w_i2h_t, b_i2h, w_i2o_t, b_i2o):
    T, B, I = x_seq.shape
    H = h0.shape[1]
    O = w_i2o_t.shape[1]
    f32, bf16 = jnp.float32, jnp.bfloat16
    Nf = H + O

    nb = 2 if (B % 16 == 0) else 1
    bb = B // nb

    out, hfin = pl.pallas_call(
        functools.partial(_rnn_kernel, isz=I, hsz=H, osz=O, bb=bb),
        grid=(nb,),
        in_specs=[
            pl.BlockSpec(memory_space=pltpu.MemorySpace.HBM),
            pl.BlockSpec((bb, H), lambda g: (g, 0)),
            pl.BlockSpec((I + H, H), lambda g: (0, 0)),
            pl.BlockSpec((1, H), lambda g: (0, 0)),
            pl.BlockSpec((I + H, O), lambda g: (0, 0)),
            pl.BlockSpec((1, O), lambda g: (0, 0)),
        ],
        out_specs=(
            pl.BlockSpec((bb, O), lambda g: (g, 0)),
            pl.BlockSpec((bb, H), lambda g: (g, 0)),
        ),
        out_shape=(
            jax.ShapeDtypeStruct((B, O), f32),
            jax.ShapeDtypeStruct((B, H), f32),
        ),
        scratch_shapes=[
            pltpu.VMEM((_NBUF, _S, bb, I), f32),
            pltpu.VMEM((_S * I, H), bf16),
            pltpu.VMEM((2 * H, H), bf16),
            pltpu.VMEM((I, Nf), bf16),
            pltpu.VMEM((H, Nf), bf16),
            pltpu.VMEM((8, Nf), f32),
            pltpu.SemaphoreType.DMA((_NBUF,)),
        ],
        compiler_params=pltpu.CompilerParams(
            dimension_semantics=("parallel",),
        ),
    )(x_seq.astype(f32), h0.astype(f32),
      w_i2h_t.astype(f32), b_i2h.astype(f32),
      w_i2o_t.astype(f32), b_i2o.astype(f32))

    return out, hfin


def kernel(x_seq, h0, w_i2h_t, b_i2h, w_i2o_t, b_i2o):
    return _rnn_fused(x_seq, h0, w_i2h_t, b_i2h, w_i2o_t, b_i2o)


# speedup vs baseline: 1.0913x; 1.0833x over previous
import functools

import jax
import jax.numpy as jnp
from jax.experimental import pallas as pl
from jax.experimental.pallas import tpu as pltpu


_S = 9
_NBUF = 4


def _rnn_kernel(x_hbm, h0_ref, wih_ref, bh_ref, wio_ref, bo_ref,
                out_ref, hfin_ref,
                xbuf, xlast, q_scr, a_scr, insem, lsem,
                *, seq_len, isz, hsz, osz, bb):
    f32 = jnp.float32
    bf16 = jnp.bfloat16
    g = pl.program_id(0)
    n_full = (seq_len - 1) // _S
    rem = (seq_len - 1) % _S
    dot = lambda u, v: jnp.dot(u, v, preferred_element_type=f32)
    db = lambda u, v: dot(u, v).astype(bf16)

    def dma_in(slot, k, rows):
        pltpu.make_async_copy(
            x_hbm.at[pl.ds(k * _S, rows), pl.ds(g * bb, bb), :],
            xbuf.at[slot, pl.ds(0, rows)], insem.at[slot]).start()

    def wait_in(slot, rows):
        pltpu.make_async_copy(
            x_hbm.at[pl.ds(0, rows), pl.ds(0, bb), :],
            xbuf.at[slot, pl.ds(0, rows)], insem.at[slot]).wait()

    n_chunks = n_full + (1 if rem else 0)
    chunk_rows = [_S] * n_full + ([rem] if rem else [])

    pltpu.make_async_copy(
        x_hbm.at[pl.ds(seq_len - 1, 1), pl.ds(g * bb, bb), :],
        xlast, lsem).start()
    for b in range(min(_NBUF, n_chunks)):
        dma_in(b, b, chunk_rows[b])

    Ab = wih_ref[isz:, :].astype(bf16)
    A2 = db(Ab, Ab)
    A4 = db(A2, A2)
    A8 = db(A4, A4)
    a_scr[...] = db(A8, Ab)

    wxhb = wih_ref[0:isz, :].astype(bf16)
    q_scr[0:isz, :] = wxhb
    q1 = db(wxhb, Ab)
    q2 = db(wxhb, A2)
    q3 = db(q1, A2)
    q4 = db(wxhb, A4)
    qs = [None, q1, q2, q3, q4, db(q1, A4), db(q2, A4), db(q3, A4), db(q4, A4)]
    for j in range(1, _S):
        q_scr[j * isz:(j + 1) * isz, :] = qs[j]

    bh = bh_ref[...]
    bhb = bh.astype(bf16)
    u1 = dot(bhb, Ab)
    u2 = dot(bhb, A2)
    u3 = dot(u1.astype(bf16), A2)
    u4 = dot(bhb, A4)
    bsum = (bh + u1 + u2 + u3 + u4
            + dot(u1.astype(bf16), A4) + dot(u2.astype(bf16), A4)
            + dot(u3.astype(bf16), A4) + dot(u4.astype(bf16), A4))

    def xdot(slot, t, j):
        x = xbuf[slot, t].astype(bf16)
        q = q_scr[j * isz:(j + 1) * isz, :]
        return jnp.dot(x, q, preferred_element_type=f32)

    h = h0_ref[...]
    for k in range(n_full):
        slot = k % _NBUF
        wait_in(slot, _S)
        s = jnp.dot(h.astype(bf16), a_scr[...], preferred_element_type=f32)
        for j in range(_S):
            s = s + xdot(slot, j, _S - 1 - j)
        h = s + bsum
        if k + _NBUF < n_chunks:
            dma_in(slot, k + _NBUF, chunk_rows[k + _NBUF])

    if rem:
        slot = n_full % _NBUF
        wait_in(slot, rem)
        for j in range(rem):
            h = (jnp.dot(h.astype(bf16), Ab, preferred_element_type=f32)
                 + xdot(slot, j, 0) + bh)

    pltpu.make_async_copy(
        x_hbm.at[pl.ds(0, 1), pl.ds(0, bb), :], xlast, lsem).wait()
    xl = xlast[0].astype(bf16)
    hb = h.astype(bf16)
    hfin_ref[...] = (jnp.dot(xl, q_scr[0:isz, :], preferred_element_type=f32)
                     + jnp.dot(hb, Ab, preferred_element_type=f32) + bh)
    logits = (jnp.dot(xl, wio_ref[0:isz, :].astype(bf16),
                      preferred_element_type=f32)
              + jnp.dot(hb, wio_ref[isz:, :].astype(bf16),
                        preferred_element_type=f32)
              + bo_ref[...])
    m = jnp.max(logits, axis=1, keepdims=True)
    sh = logits - m
    out_ref[...] = sh - jnp.log(jnp.sum(jnp.exp(sh), axis=1, keepdims=True))


@jax.jit
def _rnn_fused(x_seq, h0, w_i2h_t, b_i2h, w_i2o_t, b_i2o):
    T, B, I = x_seq.shape
    H = h0.shape[1]
    O = w_i2o_t.shape[1]
    f32, bf16 = jnp.float32, jnp.bfloat16

    nb = 2 if (B % 16 == 0) else 1
    bb = B // nb

    out, hfin = pl.pallas_call(
        functools.partial(_rnn_kernel, seq_len=T, isz=I, hsz=H, osz=O, bb=bb),
        grid=(nb,),
        in_specs=[
            pl.BlockSpec(memory_space=pltpu.MemorySpace.HBM),
            pl.BlockSpec((bb, H), lambda g: (g, 0)),
            pl.BlockSpec((I + H, H), lambda g: (0, 0)),
            pl.BlockSpec((1, H), lambda g: (0, 0)),
            pl.BlockSpec((I + H, O), lambda g: (0, 0)),
            pl.BlockSpec((1, O), lambda g: (0, 0)),
        ],
        out_specs=(
            pl.BlockSpec((bb, O), lambda g: (g, 0)),
            pl.BlockSpec((bb, H), lambda g: (g, 0)),
        ),
        out_shape=(
            jax.ShapeDtypeStruct((B, O), f32),
            jax.ShapeDtypeStruct((B, H), f32),
        ),
        scratch_shapes=[
            pltpu.VMEM((_NBUF, _S, bb, I), f32),
            pltpu.VMEM((1, bb, I), f32),
            pltpu.VMEM((_S * I, H), bf16),
            pltpu.VMEM((H, H), bf16),
            pltpu.SemaphoreType.DMA((_NBUF,)),
            pltpu.SemaphoreType.DMA,
        ],
        compiler_params=pltpu.CompilerParams(
            dimension_semantics=("parallel",),
        ),
    )(x_seq.astype(f32), h0.astype(f32),
      w_i2h_t.astype(f32), b_i2h.astype(f32),
      w_i2o_t.astype(f32), b_i2o.astype(f32))

    return out, hfin


def kernel(x_seq, h0, w_i2h_t, b_i2h, w_i2o_t, b_i2o):
    return _rnn_fused(x_seq, h0, w_i2h_t, b_i2h, w_i2o_t, b_i2o)


# speedup vs baseline: 1.0955x; 1.0038x over previous
import functools

import jax
import jax.numpy as jnp
from jax.experimental import pallas as pl
from jax.experimental.pallas import tpu as pltpu


_S = 9
_NBUF = 4


def _rnn_kernel(x_hbm, h0_ref, wih_ref, bh_ref, wio_ref, bo_ref,
                out_ref, hfin_ref,
                xbuf, xlast, q_scr, a_scr, insem, lsem,
                *, seq_len, isz, hsz, osz, bb):
    f32 = jnp.float32
    bf16 = jnp.bfloat16
    g = pl.program_id(0)
    n_full = (seq_len - 1) // _S
    rem = (seq_len - 1) % _S
    dot = lambda u, v: jnp.dot(u, v, preferred_element_type=f32)
    db = lambda u, v: dot(u, v).astype(bf16)

    def dma_in(slot, k, rows):
        for st in range(3):
            r0 = st * ((rows + 2) // 3)
            r1 = min(rows, (st + 1) * ((rows + 2) // 3))
            if r1 > r0:
                pltpu.make_async_copy(
                    x_hbm.at[pl.ds(k * _S + r0, r1 - r0), pl.ds(g * bb, bb), :],
                    xbuf.at[slot, pl.ds(r0, r1 - r0)], insem.at[slot, st]).start()

    def wait_in(slot, rows):
        for st in range(3):
            r0 = st * ((rows + 2) // 3)
            r1 = min(rows, (st + 1) * ((rows + 2) // 3))
            if r1 > r0:
                pltpu.make_async_copy(
                    x_hbm.at[pl.ds(0, r1 - r0), pl.ds(0, bb), :],
                    xbuf.at[slot, pl.ds(r0, r1 - r0)], insem.at[slot, st]).wait()

    n_chunks = n_full + (1 if rem else 0)
    chunk_rows = [_S] * n_full + ([rem] if rem else [])

    pltpu.make_async_copy(
        x_hbm.at[pl.ds(seq_len - 1, 1), pl.ds(g * bb, bb), :],
        xlast, lsem).start()
    for b in range(min(_NBUF, n_chunks)):
        dma_in(b, b, chunk_rows[b])

    Ab = wih_ref[isz:, :].astype(bf16)
    A2 = db(Ab, Ab)
    A4 = db(A2, A2)
    A8 = db(A4, A4)
    a_scr[...] = db(A8, Ab)

    wxhb = wih_ref[0:isz, :].astype(bf16)
    q_scr[0:isz, :] = wxhb
    q1 = db(wxhb, Ab)
    q2 = db(wxhb, A2)
    q3 = db(q1, A2)
    q4 = db(wxhb, A4)
    qs = [None, q1, q2, q3, q4, db(q1, A4), db(q2, A4), db(q3, A4), db(q4, A4)]
    for j in range(1, _S):
        q_scr[j * isz:(j + 1) * isz, :] = qs[j]

    bh = bh_ref[...]
    bhb = bh.astype(bf16)
    u1 = dot(bhb, Ab)
    u2 = dot(bhb, A2)
    u3 = dot(u1.astype(bf16), A2)
    u4 = dot(bhb, A4)
    bsum = (bh + u1 + u2 + u3 + u4
            + dot(u1.astype(bf16), A4) + dot(u2.astype(bf16), A4)
            + dot(u3.astype(bf16), A4) + dot(u4.astype(bf16), A4))

    def xdot(slot, t, j):
        x = xbuf[slot, t].astype(bf16)
        q = q_scr[j * isz:(j + 1) * isz, :]
        return jnp.dot(x, q, preferred_element_type=f32)

    h = h0_ref[...]
    for k in range(n_full):
        slot = k % _NBUF
        wait_in(slot, _S)
        s = jnp.dot(h.astype(bf16), a_scr[...], preferred_element_type=f32)
        for j in range(_S):
            s = s + xdot(slot, j, _S - 1 - j)
        h = s + bsum
        if k + _NBUF < n_chunks:
            dma_in(slot, k + _NBUF, chunk_rows[k + _NBUF])

    if rem:
        slot = n_full % _NBUF
        wait_in(slot, rem)
        for j in range(rem):
            h = (jnp.dot(h.astype(bf16), Ab, preferred_element_type=f32)
                 + xdot(slot, j, 0) + bh)

    pltpu.make_async_copy(
        x_hbm.at[pl.ds(0, 1), pl.ds(0, bb), :], xlast, lsem).wait()
    xl = xlast[0].astype(bf16)
    hb = h.astype(bf16)
    hfin_ref[...] = (jnp.dot(xl, q_scr[0:isz, :], preferred_element_type=f32)
                     + jnp.dot(hb, Ab, preferred_element_type=f32) + bh)
    logits = (jnp.dot(xl, wio_ref[0:isz, :].astype(bf16),
                      preferred_element_type=f32)
              + jnp.dot(hb, wio_ref[isz:, :].astype(bf16),
                        preferred_element_type=f32)
              + bo_ref[...])
    m = jnp.max(logits, axis=1, keepdims=True)
    sh = logits - m
    out_ref[...] = sh - jnp.log(jnp.sum(jnp.exp(sh), axis=1, keepdims=True))


@jax.jit
def _rnn_fused(x_seq, h0, w_i2h_t, b_i2h, w_i2o_t, b_i2o):
    T, B, I = x_seq.shape
    H = h0.shape[1]
    O = w_i2o_t.shape[1]
    f32, bf16 = jnp.float32, jnp.bfloat16

    nb = 2 if (B % 16 == 0) else 1
    bb = B // nb

    out, hfin = pl.pallas_call(
        functools.partial(_rnn_kernel, seq_len=T, isz=I, hsz=H, osz=O, bb=bb),
        grid=(nb,),
        in_specs=[
            pl.BlockSpec(memory_space=pltpu.MemorySpace.HBM),
            pl.BlockSpec((bb, H), lambda g: (g, 0)),
            pl.BlockSpec((I + H, H), lambda g: (0, 0)),
            pl.BlockSpec((1, H), lambda g: (0, 0)),
            pl.BlockSpec((I + H, O), lambda g: (0, 0)),
            pl.BlockSpec((1, O), lambda g: (0, 0)),
        ],
        out_specs=(
            pl.BlockSpec((bb, O), lambda g: (g, 0)),
            pl.BlockSpec((bb, H), lambda g: (g, 0)),
        ),
        out_shape=(
            jax.ShapeDtypeStruct((B, O), f32),
            jax.ShapeDtypeStruct((B, H), f32),
        ),
        scratch_shapes=[
            pltpu.VMEM((_NBUF, _S, bb, I), f32),
            pltpu.VMEM((1, bb, I), f32),
            pltpu.VMEM((_S * I, H), bf16),
            pltpu.VMEM((H, H), bf16),
            pltpu.SemaphoreType.DMA((_NBUF, 3)),
            pltpu.SemaphoreType.DMA,
        ],
        compiler_params=pltpu.CompilerParams(
            dimension_semantics=("parallel",),
        ),
    )(x_seq.astype(f32), h0.astype(f32),
      w_i2h_t.astype(f32), b_i2h.astype(f32),
      w_i2o_t.astype(f32), b_i2o.astype(f32))

    return out, hfin


def kernel(x_seq, h0, w_i2h_t, b_i2h, w_i2o_t, b_i2o):
    return _rnn_fused(x_seq, h0, w_i2h_t, b_i2h, w_i2o_t, b_i2o)


# speedup vs baseline: 1.1025x; 1.0064x over previous
import functools

import jax
import jax.numpy as jnp
from jax.experimental import pallas as pl
from jax.experimental.pallas import tpu as pltpu


_S = 9
_NBUF = 4


def _rnn_kernel(x_hbm, h0_ref, wih_ref, bh_ref, wio_ref, bo_ref,
                out_ref, hfin_ref,
                xbuf, xlast, q_scr, a_scr, insem, lsem,
                *, seq_len, isz, hsz, osz, bb):
    f32 = jnp.float32
    bf16 = jnp.bfloat16
    g = pl.program_id(0)
    n_full = (seq_len - 1) // _S
    rem = (seq_len - 1) % _S
    dot = lambda u, v: jnp.dot(u, v, preferred_element_type=f32)
    db = lambda u, v: dot(u, v).astype(bf16)

    def dma_in(slot, k, rows):
        pltpu.make_async_copy(
            x_hbm.at[pl.ds(k * _S, rows), pl.ds(g * bb, bb), :],
            xbuf.at[slot, pl.ds(0, rows)], insem.at[slot]).start()

    def wait_in(slot, rows):
        pltpu.make_async_copy(
            x_hbm.at[pl.ds(0, rows), pl.ds(0, bb), :],
            xbuf.at[slot, pl.ds(0, rows)], insem.at[slot]).wait()

    n_chunks = n_full + (1 if rem else 0)
    chunk_rows = [_S] * n_full + ([rem] if rem else [])

    pltpu.make_async_copy(
        x_hbm.at[pl.ds(seq_len - 1, 1), pl.ds(g * bb, bb), :],
        xlast, lsem).start()
    for b in range(min(_NBUF, n_chunks)):
        dma_in(b, b, chunk_rows[b])

    Ab = wih_ref[isz:, :].astype(bf16)
    A2 = db(Ab, Ab)
    A4 = db(A2, A2)
    A8 = db(A4, A4)
    a_scr[...] = db(A8, Ab)

    wxhb = wih_ref[0:isz, :].astype(bf16)
    q_scr[0:isz, :] = wxhb
    q1 = db(wxhb, Ab)
    q2 = db(wxhb, A2)
    q3 = db(q1, A2)
    q4 = db(wxhb, A4)
    qs = [None, q1, q2, q3, q4, db(q1, A4), db(q2, A4), db(q3, A4), db(q4, A4)]
    for j in range(1, _S):
        q_scr[j * isz:(j + 1) * isz, :] = qs[j]

    bh = bh_ref[...]
    bhb = bh.astype(bf16)
    u1 = dot(bhb, Ab)
    u2 = dot(bhb, A2)
    u3 = dot(u1.astype(bf16), A2)
    u4 = dot(bhb, A4)
    bsum = (bh + u1 + u2 + u3 + u4
            + dot(u1.astype(bf16), A4) + dot(u2.astype(bf16), A4)
            + dot(u3.astype(bf16), A4) + dot(u4.astype(bf16), A4))

    def xdot(slot, t, j):
        x = xbuf[slot, t].astype(bf16)
        q = q_scr[j * isz:(j + 1) * isz, :]
        return jnp.dot(x, q, preferred_element_type=f32)

    h = h0_ref[...]
    for k in range(n_full):
        slot = k % _NBUF
        wait_in(slot, _S)
        s = jnp.dot(h.astype(bf16), a_scr[...], preferred_element_type=f32)
        for j in range(_S):
            s = s + xdot(slot, j, _S - 1 - j)
        h = s + bsum
        if k + _NBUF < n_chunks:
            dma_in(slot, k + _NBUF, chunk_rows[k + _NBUF])

    if rem:
        slot = n_full % _NBUF
        wait_in(slot, rem)
        for j in range(rem):
            h = (jnp.dot(h.astype(bf16), Ab, preferred_element_type=f32)
                 + xdot(slot, j, 0) + bh)

    pltpu.make_async_copy(
        x_hbm.at[pl.ds(0, 1), pl.ds(0, bb), :], xlast, lsem).wait()
    xl = xlast[0].astype(bf16)
    hb = h.astype(bf16)
    hfin_ref[...] = (jnp.dot(xl, q_scr[0:isz, :], preferred_element_type=f32)
                     + jnp.dot(hb, Ab, preferred_element_type=f32) + bh)
    logits = (jnp.dot(xl, wio_ref[0:isz, :].astype(bf16),
                      preferred_element_type=f32)
              + jnp.dot(hb, wio_ref[isz:, :].astype(bf16),
                        preferred_element_type=f32)
              + bo_ref[...])
    m = jnp.max(logits, axis=1, keepdims=True)
    sh = logits - m
    out_ref[...] = sh - jnp.log(jnp.sum(jnp.exp(sh), axis=1, keepdims=True))


@jax.jit
def _rnn_fused(x_seq, h0, w_i2h_t, b_i2h, w_i2o_t, b_i2o):
    T, B, I = x_seq.shape
    H = h0.shape[1]
    O = w_i2o_t.shape[1]
    f32, bf16 = jnp.float32, jnp.bfloat16

    nb = 2 if (B % 16 == 0) else 1
    bb = B // nb

    out, hfin = pl.pallas_call(
        functools.partial(_rnn_kernel, seq_len=T, isz=I, hsz=H, osz=O, bb=bb),
        grid=(nb,),
        in_specs=[
            pl.BlockSpec(memory_space=pltpu.MemorySpace.HBM),
            pl.BlockSpec((bb, H), lambda g: (g, 0)),
            pl.BlockSpec((I + H, H), lambda g: (0, 0)),
            pl.BlockSpec((1, H), lambda g: (0, 0)),
            pl.BlockSpec((I + H, O), lambda g: (0, 0)),
            pl.BlockSpec((1, O), lambda g: (0, 0)),
        ],
        out_specs=(
            pl.BlockSpec((bb, O), lambda g: (g, 0)),
            pl.BlockSpec((bb, H), lambda g: (g, 0)),
        ),
        out_shape=(
            jax.ShapeDtypeStruct((B, O), f32),
            jax.ShapeDtypeStruct((B, H), f32),
        ),
        scratch_shapes=[
            pltpu.VMEM((_NBUF, _S, bb, I), f32),
            pltpu.VMEM((1, bb, I), f32),
            pltpu.VMEM((_S * I, H), bf16),
            pltpu.VMEM((H, H), bf16),
            pltpu.SemaphoreType.DMA((_NBUF,)),
            pltpu.SemaphoreType.DMA,
        ],
        compiler_params=pltpu.CompilerParams(
            dimension_semantics=("parallel",),
        ),
    )(x_seq.astype(f32), h0.astype(f32),
      w_i2h_t.astype(f32), b_i2h.astype(f32),
      w_i2o_t.astype(f32), b_i2o.astype(f32))

    return out, hfin


def kernel(x_seq, h0, w_i2h_t, b_i2h, w_i2o_t, b_i2o):
    return _rnn_fused(x_seq, h0, w_i2h_t, b_i2h, w_i2o_t, b_i2o)
